# Initial kernel scaffold; baseline (speedup 1.0000x reference)
import jax
import jax.numpy as jnp
from jax.experimental import pallas as pl


def kernel(x, edge_index, edge_weights, W1, b1, W2, b2, W3, b3, W4, b4, Wl, bl):
    raise NotImplementedError("write your pallas kernel here")


# speedup vs baseline: 2.0301x; 2.0301x over previous
import functools

import jax
import jax.numpy as jnp
from jax import lax
from jax.experimental import pallas as pl
from jax.experimental.pallas import tpu as pltpu
from jax.experimental.pallas import tpu_sc as plsc

N = 10000
F_IN = 128
H = 256
C = 40
R = 400
G = N // R


def _dinv_body(degp_ref, o_ref):
    deg = jnp.sum(degp_ref[...], axis=0) + 1.0
    o_ref[...] = lax.rsqrt(deg)[:, None]


def _dinv(deg_partials):
    return pl.pallas_call(
        _dinv_body,
        out_shape=jax.ShapeDtypeStruct((N, 1), jnp.float32),
    )(deg_partials)


def _t0_body(x_ref, w_ref, dinv_ref, o_ref):
    y = jnp.dot(x_ref[...], w_ref[...], preferred_element_type=jnp.float32)
    s = dinv_ref[...] * y
    o_ref[0] = s[:, :128]
    o_ref[1] = s[:, 128:]


def _t0(x, W1, dinv):
    return pl.pallas_call(
        _t0_body,
        grid=(G,),
        in_specs=[
            pl.BlockSpec((R, F_IN), lambda i: (i, 0)),
            pl.BlockSpec((F_IN, H), lambda i: (0, 0)),
            pl.BlockSpec((R, 1), lambda i: (i, 0)),
        ],
        out_specs=pl.BlockSpec((2, R, 128), lambda i: (0, i, 0)),
        out_shape=jax.ShapeDtypeStruct((2, N, 128), jnp.float32),
    )(x, W1, dinv)


def _pre_h(agg_ref, s_ref, dinv_ref, b_ref):
    t = agg_ref[...] + s_ref[...]
    cat = jnp.concatenate([t[0], t[1]], axis=1)
    pre = dinv_ref[...] * cat + b_ref[...]
    nrm = jnp.sqrt(jnp.sum(pre * pre, axis=1, keepdims=True))
    return jnp.maximum(pre / jnp.maximum(nrm, 1e-12), 0.0)


def _tmid_body(agg_ref, s_ref, dinv_ref, b_ref, w_ref, o_ref):
    h = _pre_h(agg_ref, s_ref, dinv_ref, b_ref)
    y = jnp.dot(h, w_ref[...], preferred_element_type=jnp.float32)
    s = dinv_ref[...] * y
    o_ref[0] = s[:, :128]
    o_ref[1] = s[:, 128:]


def _tmid(agg2, s2, dinv, b, W):
    return pl.pallas_call(
        _tmid_body,
        grid=(G,),
        in_specs=[
            pl.BlockSpec((2, R, 128), lambda i: (0, i, 0)),
            pl.BlockSpec((2, R, 128), lambda i: (0, i, 0)),
            pl.BlockSpec((R, 1), lambda i: (i, 0)),
            pl.BlockSpec((1, H), lambda i: (0, 0)),
            pl.BlockSpec((H, H), lambda i: (0, 0)),
        ],
        out_specs=pl.BlockSpec((2, R, 128), lambda i: (0, i, 0)),
        out_shape=jax.ShapeDtypeStruct((2, N, 128), jnp.float32),
    )(agg2, s2, dinv, b.reshape(1, H), W)


def _tfin_body(agg_ref, s_ref, dinv_ref, b_ref, wl_ref, bl_ref, o_ref):
    h = _pre_h(agg_ref, s_ref, dinv_ref, b_ref)
    o_ref[...] = (jnp.dot(h, wl_ref[...], preferred_element_type=jnp.float32)
                  + bl_ref[...])


def _tfin(agg2, s2, dinv, b, Wl, bl):
    return pl.pallas_call(
        _tfin_body,
        grid=(G,),
        in_specs=[
            pl.BlockSpec((2, R, 128), lambda i: (0, i, 0)),
            pl.BlockSpec((2, R, 128), lambda i: (0, i, 0)),
            pl.BlockSpec((R, 1), lambda i: (i, 0)),
            pl.BlockSpec((1, H), lambda i: (0, 0)),
            pl.BlockSpec((H, C), lambda i: (0, 0)),
            pl.BlockSpec((1, C), lambda i: (0, 0)),
        ],
        out_specs=pl.BlockSpec((R, C), lambda i: (i, 0)),
        out_shape=jax.ShapeDtypeStruct((N, C), jnp.float32),
    )(agg2, s2, dinv, b.reshape(1, H), Wl, bl.reshape(1, C))


def _deg_partials(dst, ew):
    p = jax.ops.segment_sum(ew, dst, num_segments=N)
    return jnp.zeros((32, N), jnp.float32).at[0].set(p)


def _spmm(s2, src, dst, ew):
    s = jnp.concatenate([s2[0], s2[1]], axis=1)
    agg = jax.ops.segment_sum(s[src] * ew[:, None], dst, num_segments=N)
    return jnp.stack([agg[:, :128], agg[:, 128:]])


def kernel(x, edge_index, edge_weights, W1, b1, W2, b2, W3, b3, W4, b4, Wl, bl):
    src = edge_index[0]
    dst = edge_index[1]
    ew = edge_weights

    dinv = _dinv(_deg_partials(dst, ew))
    s2 = _t0(x, W1, dinv)
    for b, W in ((b1, W2), (b2, W3), (b3, W4)):
        agg2 = _spmm(s2, src, dst, ew)
        s2 = _tmid(agg2, s2, dinv, b, W)
    agg2 = _spmm(s2, src, dst, ew)
    return _tfin(agg2, s2, dinv, b4, Wl, bl)


# speedup vs baseline: 5.4574x; 2.6882x over previous
import functools

import jax
import jax.numpy as jnp
from jax import lax
from jax.experimental import pallas as pl
from jax.experimental.pallas import tpu as pltpu
from jax.experimental.pallas import tpu_sc as plsc

N = 10000
F_IN = 128
H = 256
C = 40
R = 400
G = N // R


def _dinv_body(degp_ref, o_ref):
    deg = jnp.sum(degp_ref[...], axis=0) + 1.0
    o_ref[...] = lax.rsqrt(deg)[:, None]


def _dinv(deg_partials):
    return pl.pallas_call(
        _dinv_body,
        out_shape=jax.ShapeDtypeStruct((N, 1), jnp.float32),
    )(deg_partials)


def _t0_body(x_ref, w_ref, dinv_ref, o_ref):
    y = jnp.dot(x_ref[...], w_ref[...], preferred_element_type=jnp.float32)
    s = dinv_ref[...] * y
    o_ref[0] = s[:, :128]
    o_ref[1] = s[:, 128:]


def _t0(x, W1, dinv):
    return pl.pallas_call(
        _t0_body,
        grid=(G,),
        in_specs=[
            pl.BlockSpec((R, F_IN), lambda i: (i, 0)),
            pl.BlockSpec((F_IN, H), lambda i: (0, 0)),
            pl.BlockSpec((R, 1), lambda i: (i, 0)),
        ],
        out_specs=pl.BlockSpec((2, R, 128), lambda i: (0, i, 0)),
        out_shape=jax.ShapeDtypeStruct((2, N, 128), jnp.float32),
    )(x, W1, dinv)


def _pre_h(agg_ref, s_ref, dinv_ref, b_ref):
    t = agg_ref[...] + s_ref[...]
    cat = jnp.concatenate([t[0], t[1]], axis=1)
    pre = dinv_ref[...] * cat + b_ref[...]
    nrm = jnp.sqrt(jnp.sum(pre * pre, axis=1, keepdims=True))
    return jnp.maximum(pre / jnp.maximum(nrm, 1e-12), 0.0)


def _tmid_body(agg_ref, s_ref, dinv_ref, b_ref, w_ref, o_ref):
    h = _pre_h(agg_ref, s_ref, dinv_ref, b_ref)
    y = jnp.dot(h, w_ref[...], preferred_element_type=jnp.float32)
    s = dinv_ref[...] * y
    o_ref[0] = s[:, :128]
    o_ref[1] = s[:, 128:]


def _tmid(agg2, s2, dinv, b, W):
    return pl.pallas_call(
        _tmid_body,
        grid=(G,),
        in_specs=[
            pl.BlockSpec((2, R, 128), lambda i: (0, i, 0)),
            pl.BlockSpec((2, R, 128), lambda i: (0, i, 0)),
            pl.BlockSpec((R, 1), lambda i: (i, 0)),
            pl.BlockSpec((1, H), lambda i: (0, 0)),
            pl.BlockSpec((H, H), lambda i: (0, 0)),
        ],
        out_specs=pl.BlockSpec((2, R, 128), lambda i: (0, i, 0)),
        out_shape=jax.ShapeDtypeStruct((2, N, 128), jnp.float32),
    )(agg2, s2, dinv, b.reshape(1, H), W)


def _tfin_body(agg_ref, s_ref, dinv_ref, b_ref, wl_ref, bl_ref, o_ref):
    h = _pre_h(agg_ref, s_ref, dinv_ref, b_ref)
    o_ref[...] = (jnp.dot(h, wl_ref[...], preferred_element_type=jnp.float32)
                  + bl_ref[...])


def _tfin(agg2, s2, dinv, b, Wl, bl):
    return pl.pallas_call(
        _tfin_body,
        grid=(G,),
        in_specs=[
            pl.BlockSpec((2, R, 128), lambda i: (0, i, 0)),
            pl.BlockSpec((2, R, 128), lambda i: (0, i, 0)),
            pl.BlockSpec((R, 1), lambda i: (i, 0)),
            pl.BlockSpec((1, H), lambda i: (0, 0)),
            pl.BlockSpec((H, C), lambda i: (0, 0)),
            pl.BlockSpec((1, C), lambda i: (0, 0)),
        ],
        out_specs=pl.BlockSpec((R, C), lambda i: (i, 0)),
        out_shape=jax.ShapeDtypeStruct((N, C), jnp.float32),
    )(agg2, s2, dinv, b.reshape(1, H), Wl, bl.reshape(1, C))


EPAD = 321536
DPW = EPAD // 32
DCH = 64
NDCH = DPW // DCH


def _sc_mesh():
    return plsc.VectorSubcoreMesh(core_axis_name="c", subcore_axis_name="s")


@functools.partial(
    pl.kernel,
    out_type=jax.ShapeDtypeStruct((32, N), jnp.float32),
    mesh=_sc_mesh(),
    compiler_params=pltpu.CompilerParams(needs_layout_passes=False),
    scratch_types=[
        pltpu.VMEM((N,), jnp.float32),
        pltpu.VMEM((DCH,), jnp.int32),
        pltpu.VMEM((DCH,), jnp.float32),
    ],
)
def _deg_sc(dst_hbm, ew_hbm, out_hbm, dacc, dbuf, ebuf):
    cid = lax.axis_index("c")
    sid = lax.axis_index("s")
    wid = sid * 2 + cid
    zero = jnp.zeros((16,), jnp.float32)

    def zbody(i, c):
        dacc[pl.ds(i * 16, 16)] = zero
        return c

    lax.fori_loop(0, N // 16, zbody, 0)
    base = wid * DPW

    def cbody(i, c):
        off = base + i * DCH
        pltpu.sync_copy(dst_hbm.at[pl.ds(off, DCH)], dbuf)
        pltpu.sync_copy(ew_hbm.at[pl.ds(off, DCH)], ebuf)
        for q in range(DCH // 16):
            idx = dbuf[pl.ds(q * 16, 16)]
            val = ebuf[pl.ds(q * 16, 16)]
            plsc.addupdate_scatter(dacc, [idx], val)
        return c

    lax.fori_loop(0, NDCH, cbody, 0)
    pltpu.sync_copy(dacc, out_hbm.at[wid])


def _deg_partials(dst, ew):
    return _deg_sc(dst, ew)


SPS = EPAD // 16
SCH = 128
NSCH = SPS // SCH
NPAD = 10240
RPS = NPAD // 16


@functools.partial(
    pl.kernel,
    out_type=jax.ShapeDtypeStruct((2, NPAD, 128), jnp.float32),
    mesh=_sc_mesh(),
    compiler_params=pltpu.CompilerParams(needs_layout_passes=False),
    scratch_types=[
        pltpu.VMEM((SCH, 128), jnp.float32),
        pltpu.VMEM((SCH,), jnp.int32),
        pltpu.VMEM((2, SCH), jnp.int32),
        pltpu.VMEM((SCH,), jnp.float32),
        pltpu.VMEM((128, 128), jnp.float32),
        pltpu.VMEM_SHARED((NPAD, 128), jnp.float32),
        pltpu.SemaphoreType.DMA,
        pltpu.SemaphoreType.DMA,
    ],
)
def _spmm_sc(s2f_hbm, src_hbm, dst_hbm, ew_hbm, out_hbm,
             rows, gidx, didx, ebuf, zbuf, acc, gsem, ssem):
    cid = lax.axis_index("c")
    sid = lax.axis_index("s")
    zero = jnp.zeros((16,), jnp.float32)

    def zb(i, c):
        for j in range(8):
            zbuf[i, pl.ds(j * 16, 16)] = zero
        return c

    lax.fori_loop(0, 128, zb, 0)
    base_r = sid * RPS
    for q in range(5):
        pltpu.sync_copy(zbuf, acc.at[pl.ds(base_r + q * 128, 128)])
    plsc.subcore_barrier()

    cN = cid * N
    base_e = sid * SPS

    def body(i, c):
        off = base_e + i * SCH
        pltpu.sync_copy(src_hbm.at[pl.ds(off, SCH)], gidx)
        pltpu.sync_copy(dst_hbm.at[pl.ds(off, SCH)], didx.at[0])
        pltpu.sync_copy(ew_hbm.at[pl.ds(off, SCH)], ebuf)
        for q in range(SCH // 16):
            gidx[pl.ds(q * 16, 16)] = gidx[pl.ds(q * 16, 16)] + cN
        pltpu.async_copy(s2f_hbm.at[gidx], rows, gsem).wait()

        def scale(e, c2):
            ewv = plsc.load_gather(ebuf, [jnp.full((16,), e, jnp.int32)])
            for j in range(8):
                sl = pl.ds(j * 16, 16)
                rows[e, sl] = rows[e, sl] * ewv
            return c2

        lax.fori_loop(0, SCH, scale, 0)
        pltpu.async_copy(rows, acc.at[didx.at[0]], ssem, add=True).wait()
        return c

    lax.fori_loop(0, NSCH, body, 0)
    plsc.subcore_barrier()
    pltpu.sync_copy(acc.at[pl.ds(base_r, RPS)],
                    out_hbm.at[cid, pl.ds(base_r, RPS)])


def _spmm(s2, src, dst, ew):
    return _spmm_sc(s2.reshape(2 * N, 128), src, dst, ew)


def kernel(x, edge_index, edge_weights, W1, b1, W2, b2, W3, b3, W4, b4, Wl, bl):
    e = edge_index.shape[1]
    pad = EPAD - e
    src = jnp.concatenate([edge_index[0], jnp.zeros((pad,), jnp.int32)])
    dst = jnp.concatenate([edge_index[1], jnp.zeros((pad,), jnp.int32)])
    ew = jnp.concatenate([edge_weights, jnp.zeros((pad,), jnp.float32)])

    dinv = _dinv(_deg_partials(dst, ew))
    s2 = _t0(x, W1, dinv)
    for b, W in ((b1, W2), (b2, W3), (b3, W4)):
        agg2 = _spmm(s2, src, dst, ew)
        s2 = _tmid(agg2, s2, dinv, b, W)
    agg2 = _spmm(s2, src, dst, ew)
    return _tfin(agg2, s2, dinv, b4, Wl, bl)


# speedup vs baseline: 11.1340x; 2.0402x over previous
import functools

import jax
import jax.numpy as jnp
from jax import lax
from jax.experimental import pallas as pl
from jax.experimental.pallas import tpu as pltpu
from jax.experimental.pallas import tpu_sc as plsc

N = 10000
F_IN = 128
H = 256
C = 40
R = 400
G = N // R


def _dinv_body(degp_ref, o_ref):
    deg = jnp.sum(degp_ref[...], axis=0) + 1.0
    o_ref[...] = lax.rsqrt(deg)[:, None]


def _dinv(deg_partials):
    return pl.pallas_call(
        _dinv_body,
        out_shape=jax.ShapeDtypeStruct((N, 1), jnp.float32),
    )(deg_partials)


def _t0_body(x_ref, w_ref, dinv_ref, o_ref):
    y = jnp.dot(x_ref[...], w_ref[...], preferred_element_type=jnp.float32)
    s = dinv_ref[...] * y
    o_ref[0] = s[:, :128]
    o_ref[1] = s[:, 128:]


def _t0(x, W1, dinv):
    return pl.pallas_call(
        _t0_body,
        grid=(G,),
        in_specs=[
            pl.BlockSpec((R, F_IN), lambda i: (i, 0)),
            pl.BlockSpec((F_IN, H), lambda i: (0, 0)),
            pl.BlockSpec((R, 1), lambda i: (i, 0)),
        ],
        out_specs=pl.BlockSpec((2, R, 128), lambda i: (0, i, 0)),
        out_shape=jax.ShapeDtypeStruct((2, N, 128), jnp.float32),
    )(x, W1, dinv)


def _pre_h(agg_ref, s_ref, dinv_ref, b_ref):
    t = agg_ref[...] + s_ref[...]
    cat = jnp.concatenate([t[0], t[1]], axis=1)
    pre = dinv_ref[...] * cat + b_ref[...]
    nrm = jnp.sqrt(jnp.sum(pre * pre, axis=1, keepdims=True))
    return jnp.maximum(pre / jnp.maximum(nrm, 1e-12), 0.0)


def _tmid_body(agg_ref, s_ref, dinv_ref, b_ref, w_ref, o_ref):
    h = _pre_h(agg_ref, s_ref, dinv_ref, b_ref)
    y = jnp.dot(h, w_ref[...], preferred_element_type=jnp.float32)
    s = dinv_ref[...] * y
    o_ref[0] = s[:, :128]
    o_ref[1] = s[:, 128:]


def _tmid(agg2, s2, dinv, b, W):
    return pl.pallas_call(
        _tmid_body,
        grid=(G,),
        in_specs=[
            pl.BlockSpec((2, R, 128), lambda i: (0, i, 0)),
            pl.BlockSpec((2, R, 128), lambda i: (0, i, 0)),
            pl.BlockSpec((R, 1), lambda i: (i, 0)),
            pl.BlockSpec((1, H), lambda i: (0, 0)),
            pl.BlockSpec((H, H), lambda i: (0, 0)),
        ],
        out_specs=pl.BlockSpec((2, R, 128), lambda i: (0, i, 0)),
        out_shape=jax.ShapeDtypeStruct((2, N, 128), jnp.float32),
    )(agg2, s2, dinv, b.reshape(1, H), W)


def _tfin_body(agg_ref, s_ref, dinv_ref, b_ref, wl_ref, bl_ref, o_ref):
    h = _pre_h(agg_ref, s_ref, dinv_ref, b_ref)
    o_ref[...] = (jnp.dot(h, wl_ref[...], preferred_element_type=jnp.float32)
                  + bl_ref[...])


def _tfin(agg2, s2, dinv, b, Wl, bl):
    return pl.pallas_call(
        _tfin_body,
        grid=(G,),
        in_specs=[
            pl.BlockSpec((2, R, 128), lambda i: (0, i, 0)),
            pl.BlockSpec((2, R, 128), lambda i: (0, i, 0)),
            pl.BlockSpec((R, 1), lambda i: (i, 0)),
            pl.BlockSpec((1, H), lambda i: (0, 0)),
            pl.BlockSpec((H, C), lambda i: (0, 0)),
            pl.BlockSpec((1, C), lambda i: (0, 0)),
        ],
        out_specs=pl.BlockSpec((R, C), lambda i: (i, 0)),
        out_shape=jax.ShapeDtypeStruct((N, C), jnp.float32),
    )(agg2, s2, dinv, b.reshape(1, H), Wl, bl.reshape(1, C))


EPAD = 322560
DPW = EPAD // 32
DCH = 96
NDCH = DPW // DCH


def _sc_mesh():
    return plsc.VectorSubcoreMesh(core_axis_name="c", subcore_axis_name="s")


@functools.partial(
    pl.kernel,
    out_type=jax.ShapeDtypeStruct((32, N), jnp.float32),
    mesh=_sc_mesh(),
    compiler_params=pltpu.CompilerParams(needs_layout_passes=False),
    scratch_types=[
        pltpu.VMEM((N,), jnp.float32),
        pltpu.VMEM((DCH,), jnp.int32),
        pltpu.VMEM((DCH,), jnp.float32),
    ],
)
def _deg_sc(dst_hbm, ew_hbm, out_hbm, dacc, dbuf, ebuf):
    cid = lax.axis_index("c")
    sid = lax.axis_index("s")
    wid = sid * 2 + cid
    zero = jnp.zeros((16,), jnp.float32)

    def zbody(i, c):
        dacc[pl.ds(i * 16, 16)] = zero
        return c

    lax.fori_loop(0, N // 16, zbody, 0)
    base = wid * DPW

    def cbody(i, c):
        off = base + i * DCH
        pltpu.sync_copy(dst_hbm.at[pl.ds(off, DCH)], dbuf)
        pltpu.sync_copy(ew_hbm.at[pl.ds(off, DCH)], ebuf)
        for q in range(DCH // 16):
            idx = dbuf[pl.ds(q * 16, 16)]
            val = ebuf[pl.ds(q * 16, 16)]
            plsc.addupdate_scatter(dacc, [idx], val)
        return c

    lax.fori_loop(0, NDCH, cbody, 0)
    pltpu.sync_copy(dacc, out_hbm.at[wid])


def _deg_partials(dst, ew):
    return _deg_sc(dst, ew)


SPS = EPAD // 16
SCH = 112
NSCH = SPS // SCH
NPAD = 10240
RPS = NPAD // 16


@functools.partial(
    pl.kernel,
    out_type=jax.ShapeDtypeStruct((2, NPAD, 128), jnp.float32),
    mesh=_sc_mesh(),
    compiler_params=pltpu.CompilerParams(needs_layout_passes=False),
    scratch_types=[
        pltpu.VMEM((3, SCH, 128), jnp.float32),
        pltpu.VMEM((6, SCH), jnp.int32),
        pltpu.VMEM((6, SCH), jnp.int32),
        pltpu.VMEM((6, SCH), jnp.float32),
        pltpu.VMEM_SHARED((NPAD, 128), jnp.float32),
        pltpu.SemaphoreType.DMA,
        pltpu.SemaphoreType.DMA,
        pltpu.SemaphoreType.DMA,
        pltpu.SemaphoreType.DMA,
        pltpu.SemaphoreType.DMA,
        pltpu.SemaphoreType.DMA,
        pltpu.SemaphoreType.DMA,
        pltpu.SemaphoreType.DMA,
        pltpu.SemaphoreType.DMA,
    ],
)
def _spmm_sc(s2f_hbm, src_hbm, dst_hbm, ew_hbm, out_hbm,
             rows, gidxb, didxb, ewwb, acc,
             gs0, gs1, gs2, ss0, ss1, ss2, is0, is1, is2):
    cid = lax.axis_index("c")
    sid = lax.axis_index("s")
    gsem = (gs0, gs1, gs2)
    ssem = (ss0, ss1, ss2)
    isem = (is0, is1, is2)
    zero = jnp.zeros((16,), jnp.float32)
    cN = cid * N
    base_r = sid * RPS
    base_e = sid * SPS

    def zb(i, c):
        for j in range(8):
            rows[0, i, pl.ds(j * 16, 16)] = zero
        return c

    lax.fori_loop(0, SCH, zb, 0)
    for q in range(5):
        pltpu.sync_copy(rows.at[0], acc.at[pl.ds(base_r + q * SCH, SCH)])
    pltpu.sync_copy(rows.at[0, pl.ds(0, 80)],
                    acc.at[pl.ds(base_r + 5 * SCH, 80)])
    plsc.subcore_barrier()

    def idx_start(i, islot, sb):
        off = base_e + i * SCH
        pltpu.async_copy(src_hbm.at[pl.ds(off, SCH)], gidxb.at[islot],
                         isem[sb])
        pltpu.async_copy(dst_hbm.at[pl.ds(off, SCH)], didxb.at[islot],
                         isem[sb])
        pltpu.async_copy(ew_hbm.at[pl.ds(off, SCH)], ewwb.at[islot],
                         isem[sb])

    def idx_wait(i, islot, sb):
        off = base_e + i * SCH
        pltpu.make_async_copy(src_hbm.at[pl.ds(off, SCH)], gidxb.at[islot],
                              isem[sb]).wait()
        pltpu.make_async_copy(dst_hbm.at[pl.ds(off, SCH)], didxb.at[islot],
                              isem[sb]).wait()
        pltpu.make_async_copy(ew_hbm.at[pl.ds(off, SCH)], ewwb.at[islot],
                              isem[sb]).wait()

    def addcn(islot):
        for j in range(SCH // 16):
            sl = pl.ds(j * 16, 16)
            gidxb[islot, sl] = gidxb[islot, sl] + cN

    def gather_start(islot, b):
        pltpu.async_copy(s2f_hbm.at[gidxb.at[islot]], rows.at[b], gsem[b])

    def gather_wait(islot, b):
        pltpu.make_async_copy(s2f_hbm.at[gidxb.at[islot]], rows.at[b],
                              gsem[b]).wait()

    def scatter_start(islot, b):
        pltpu.async_copy(rows.at[b], acc.at[didxb.at[islot]], ssem[b],
                         add=True)

    def scatter_wait(islot, b):
        pltpu.make_async_copy(rows.at[b], acc.at[didxb.at[islot]],
                              ssem[b]).wait()

    def scale(islot, b):
        def sbody(e, c):
            ewv = plsc.load_gather(
                ewwb, [jnp.full((16,), islot, jnp.int32),
                       jnp.full((16,), e, jnp.int32)])
            for j in range(8):
                sl = pl.ds(j * 16, 16)
                rows[b, e, sl] = rows[b, e, sl] * ewv
            return c

        lax.fori_loop(0, SCH, sbody, 0)

    for b in range(3):
        idx_start(b, b, b)
    for b in range(2):
        idx_wait(b, b, b)
        addcn(b)
        gather_start(b, b)
    for b in range(3):
        gather_wait(b, b)
        idx_start(b + 3, b + 3, b)
        scale(b, b)
        scatter_start(b, b)
        if b > 0:
            scatter_wait(b - 1, b - 1)
        b2 = (b + 2) % 3
        i2 = b + 2
        if b == 0:
            idx_wait(2, 2, 2)
            addcn(2)
            gather_start(2, 2)
        else:
            idx_wait(i2, i2, b2)
            addcn(i2)
            gather_start(i2, b2)

    def group(g, c):
        i0 = g * 3
        gpar = g & 1
        npar = 1 - gpar
        for b in range(3):
            i = i0 + b
            islot = b + 3 * gpar
            gather_wait(islot, b)
            idx_start(i + 3, b + 3 * npar, b)
            scale(islot, b)
            scatter_start(islot, b)
            b2 = (b + 2) % 3
            scatter_wait((b2 + 3 * gpar) if b > 0 else (b2 + 3 * npar), b2)
            if b < 1:
                islot2 = b + 2 + 3 * gpar
            else:
                islot2 = (b + 2) % 3 + 3 * npar
            idx_wait(i + 2, islot2, b2)
            addcn(islot2)
            gather_start(islot2, b2)
        return c

    lax.fori_loop(1, NSCH // 3 - 1, group, 0)

    gather_wait(3, 0)
    scale(3, 0)
    scatter_start(3, 0)
    scatter_wait(2 + 3 * 0, 2)
    idx_wait(NSCH - 1, 5, 2)
    addcn(5)
    gather_start(5, 2)
    for b in (1, 2):
        gather_wait(3 + b, b)
        scale(3 + b, b)
        scatter_start(3 + b, b)
        scatter_wait(3 + b - 1, b - 1)
    scatter_wait(5, 2)

    plsc.subcore_barrier()
    pltpu.sync_copy(acc.at[pl.ds(base_r, RPS)],
                    out_hbm.at[cid, pl.ds(base_r, RPS)])


def _spmm(s2, src, dst, ew):
    return _spmm_sc(s2.reshape(2 * N, 128), src, dst, ew)


def kernel(x, edge_index, edge_weights, W1, b1, W2, b2, W3, b3, W4, b4, Wl, bl):
    e = edge_index.shape[1]
    pad = EPAD - e
    src = jnp.concatenate([edge_index[0], jnp.zeros((pad,), jnp.int32)])
    dst = jnp.concatenate([edge_index[1], jnp.zeros((pad,), jnp.int32)])
    ew = jnp.concatenate([edge_weights, jnp.zeros((pad,), jnp.float32)])

    dinv = _dinv(_deg_partials(dst, ew))
    s2 = _t0(x, W1, dinv)
    for b, W in ((b1, W2), (b2, W3), (b3, W4)):
        agg2 = _spmm(s2, src, dst, ew)
        s2 = _tmid(agg2, s2, dinv, b, W)
    agg2 = _spmm(s2, src, dst, ew)
    return _tfin(agg2, s2, dinv, b4, Wl, bl)


# speedup vs baseline: 13.1077x; 1.1773x over previous
import functools

import jax
import jax.numpy as jnp
from jax import lax
from jax.experimental import pallas as pl
from jax.experimental.pallas import tpu as pltpu
from jax.experimental.pallas import tpu_sc as plsc

N = 10000
F_IN = 128
H = 256
C = 40
R = 400
G = N // R


def _dinv_body(degp_ref, o_ref):
    deg = jnp.sum(degp_ref[...], axis=0) + 1.0
    o_ref[...] = lax.rsqrt(deg)[:, None]


def _dinv(deg_partials):
    return pl.pallas_call(
        _dinv_body,
        out_shape=jax.ShapeDtypeStruct((N, 1), jnp.float32),
    )(deg_partials)


def _t0_body(x_ref, w_ref, dinv_ref, o_ref):
    y = jnp.dot(x_ref[...], w_ref[...], preferred_element_type=jnp.float32)
    s = dinv_ref[...] * y
    o_ref[0] = s[:, :128]
    o_ref[1] = s[:, 128:]


def _t0(x, W1, dinv):
    return pl.pallas_call(
        _t0_body,
        grid=(G,),
        in_specs=[
            pl.BlockSpec((R, F_IN), lambda i: (i, 0)),
            pl.BlockSpec((F_IN, H), lambda i: (0, 0)),
            pl.BlockSpec((R, 1), lambda i: (i, 0)),
        ],
        out_specs=pl.BlockSpec((2, R, 128), lambda i: (0, i, 0)),
        out_shape=jax.ShapeDtypeStruct((2, N, 128), jnp.float32),
    )(x, W1, dinv)


def _pre_h(agg_ref, s_ref, dinv_ref, b_ref):
    t = agg_ref[...] + s_ref[...]
    cat = jnp.concatenate([t[0], t[1]], axis=1)
    pre = dinv_ref[...] * cat + b_ref[...]
    nrm = jnp.sqrt(jnp.sum(pre * pre, axis=1, keepdims=True))
    return jnp.maximum(pre / jnp.maximum(nrm, 1e-12), 0.0)


def _tmid_body(agg_ref, s_ref, dinv_ref, b_ref, w_ref, o_ref):
    h = _pre_h(agg_ref, s_ref, dinv_ref, b_ref)
    y = jnp.dot(h, w_ref[...], preferred_element_type=jnp.float32)
    s = dinv_ref[...] * y
    o_ref[0] = s[:, :128]
    o_ref[1] = s[:, 128:]


def _tmid(agg2, s2, dinv, b, W):
    return pl.pallas_call(
        _tmid_body,
        grid=(G,),
        in_specs=[
            pl.BlockSpec((2, R, 128), lambda i: (0, i, 0)),
            pl.BlockSpec((2, R, 128), lambda i: (0, i, 0)),
            pl.BlockSpec((R, 1), lambda i: (i, 0)),
            pl.BlockSpec((1, H), lambda i: (0, 0)),
            pl.BlockSpec((H, H), lambda i: (0, 0)),
        ],
        out_specs=pl.BlockSpec((2, R, 128), lambda i: (0, i, 0)),
        out_shape=jax.ShapeDtypeStruct((2, N, 128), jnp.float32),
    )(agg2, s2, dinv, b.reshape(1, H), W)


def _tfin_body(agg_ref, s_ref, dinv_ref, b_ref, wl_ref, bl_ref, o_ref):
    h = _pre_h(agg_ref, s_ref, dinv_ref, b_ref)
    o_ref[...] = (jnp.dot(h, wl_ref[...], preferred_element_type=jnp.float32)
                  + bl_ref[...])


def _tfin(agg2, s2, dinv, b, Wl, bl):
    return pl.pallas_call(
        _tfin_body,
        grid=(G,),
        in_specs=[
            pl.BlockSpec((2, R, 128), lambda i: (0, i, 0)),
            pl.BlockSpec((2, R, 128), lambda i: (0, i, 0)),
            pl.BlockSpec((R, 1), lambda i: (i, 0)),
            pl.BlockSpec((1, H), lambda i: (0, 0)),
            pl.BlockSpec((H, C), lambda i: (0, 0)),
            pl.BlockSpec((1, C), lambda i: (0, 0)),
        ],
        out_specs=pl.BlockSpec((R, C), lambda i: (i, 0)),
        out_shape=jax.ShapeDtypeStruct((N, C), jnp.float32),
    )(agg2, s2, dinv, b.reshape(1, H), Wl, bl.reshape(1, C))


EPAD = 322560
DPW = EPAD // 32
DCH = 96
NDCH = DPW // DCH


def _sc_mesh():
    return plsc.VectorSubcoreMesh(core_axis_name="c", subcore_axis_name="s")


@functools.partial(
    pl.kernel,
    out_type=jax.ShapeDtypeStruct((32, N), jnp.float32),
    mesh=_sc_mesh(),
    compiler_params=pltpu.CompilerParams(needs_layout_passes=False),
    scratch_types=[
        pltpu.VMEM((N,), jnp.float32),
        pltpu.VMEM((DCH,), jnp.int32),
        pltpu.VMEM((DCH,), jnp.float32),
    ],
)
def _deg_sc(dst_hbm, ew_hbm, out_hbm, dacc, dbuf, ebuf):
    cid = lax.axis_index("c")
    sid = lax.axis_index("s")
    wid = sid * 2 + cid
    zero = jnp.zeros((16,), jnp.float32)

    def zbody(i, c):
        dacc[pl.ds(i * 16, 16)] = zero
        return c

    lax.fori_loop(0, N // 16, zbody, 0)
    base = wid * DPW

    def cbody(i, c):
        off = base + i * DCH
        pltpu.sync_copy(dst_hbm.at[pl.ds(off, DCH)], dbuf)
        pltpu.sync_copy(ew_hbm.at[pl.ds(off, DCH)], ebuf)
        for q in range(DCH // 16):
            idx = dbuf[pl.ds(q * 16, 16)]
            val = ebuf[pl.ds(q * 16, 16)]
            plsc.addupdate_scatter(dacc, [idx], val)
        return c

    lax.fori_loop(0, NDCH, cbody, 0)
    pltpu.sync_copy(dacc, out_hbm.at[wid])


def _deg_partials(dst, ew):
    return _deg_sc(dst, ew)


SPS = EPAD // 16
SCH = 112
NSCH = SPS // SCH
NPAD = 10240
RPS = NPAD // 16


@functools.partial(
    pl.kernel,
    out_type=jax.ShapeDtypeStruct((2, NPAD, 128), jnp.float32),
    mesh=_sc_mesh(),
    compiler_params=pltpu.CompilerParams(needs_layout_passes=False),
    scratch_types=[
        pltpu.VMEM((3, SCH, 128), jnp.float32),
        pltpu.VMEM((6, SCH), jnp.int32),
        pltpu.VMEM((6, SCH), jnp.int32),
        pltpu.VMEM((6, SCH), jnp.float32),
        pltpu.VMEM_SHARED((NPAD, 128), jnp.float32),
        pltpu.SemaphoreType.DMA,
        pltpu.SemaphoreType.DMA,
        pltpu.SemaphoreType.DMA,
        pltpu.SemaphoreType.DMA,
        pltpu.SemaphoreType.DMA,
        pltpu.SemaphoreType.DMA,
        pltpu.SemaphoreType.DMA,
        pltpu.SemaphoreType.DMA,
        pltpu.SemaphoreType.DMA,
    ],
)
def _spmm_sc(s2f_hbm, src_hbm, dst_hbm, ew_hbm, out_hbm,
             rows, gidxb, didxb, ewwb, acc,
             gs0, gs1, gs2, ss0, ss1, ss2, is0, is1, is2):
    cid = lax.axis_index("c")
    sid = lax.axis_index("s")
    gsem = (gs0, gs1, gs2)
    ssem = (ss0, ss1, ss2)
    isem = (is0, is1, is2)
    zero = jnp.zeros((16,), jnp.float32)
    cN = cid * N
    base_r = sid * RPS
    base_e = sid * SPS

    def zb(i, c):
        for j in range(8):
            rows[0, i, pl.ds(j * 16, 16)] = zero
        return c

    lax.fori_loop(0, SCH, zb, 0)
    for q in range(5):
        pltpu.sync_copy(rows.at[0], acc.at[pl.ds(base_r + q * SCH, SCH)])
    pltpu.sync_copy(rows.at[0, pl.ds(0, 80)],
                    acc.at[pl.ds(base_r + 5 * SCH, 80)])
    plsc.subcore_barrier()

    def idx_start(i, islot, sb):
        off = base_e + i * SCH
        pltpu.async_copy(src_hbm.at[pl.ds(off, SCH)], gidxb.at[islot],
                         isem[sb])
        pltpu.async_copy(dst_hbm.at[pl.ds(off, SCH)], didxb.at[islot],
                         isem[sb])
        pltpu.async_copy(ew_hbm.at[pl.ds(off, SCH)], ewwb.at[islot],
                         isem[sb])

    def idx_wait(i, islot, sb):
        off = base_e + i * SCH
        pltpu.make_async_copy(src_hbm.at[pl.ds(off, SCH)], gidxb.at[islot],
                              isem[sb]).wait()
        pltpu.make_async_copy(dst_hbm.at[pl.ds(off, SCH)], didxb.at[islot],
                              isem[sb]).wait()
        pltpu.make_async_copy(ew_hbm.at[pl.ds(off, SCH)], ewwb.at[islot],
                              isem[sb]).wait()

    def addcn(islot):
        for j in range(SCH // 16):
            sl = pl.ds(j * 16, 16)
            gidxb[islot, sl] = gidxb[islot, sl] + cN

    def gather_start(islot, b):
        pltpu.async_copy(s2f_hbm.at[gidxb.at[islot]], rows.at[b], gsem[b])

    def gather_wait(islot, b):
        pltpu.make_async_copy(s2f_hbm.at[gidxb.at[islot]], rows.at[b],
                              gsem[b]).wait()

    def scatter_start(islot, b):
        pltpu.async_copy(rows.at[b], acc.at[didxb.at[islot]], ssem[b],
                         add=True)

    def scatter_wait(islot, b):
        pltpu.make_async_copy(rows.at[b], acc.at[didxb.at[islot]],
                              ssem[b]).wait()

    def scale(islot, b):
        @functools.partial(plsc.parallel_loop, 0, SCH, unroll=4)
        def sbody(e):
            ewv = plsc.load_gather(
                ewwb, [jnp.full((16,), islot, jnp.int32),
                       jnp.full((16,), e, jnp.int32)])
            for j in range(8):
                sl = pl.ds(j * 16, 16)
                rows[b, e, sl] = rows[b, e, sl] * ewv

    for b in range(3):
        idx_start(b, b, b)
    for b in range(2):
        idx_wait(b, b, b)
        addcn(b)
        gather_start(b, b)
    for b in range(3):
        gather_wait(b, b)
        idx_start(b + 3, b + 3, b)
        scale(b, b)
        scatter_start(b, b)
        if b > 0:
            scatter_wait(b - 1, b - 1)
        b2 = (b + 2) % 3
        i2 = b + 2
        if b == 0:
            idx_wait(2, 2, 2)
            addcn(2)
            gather_start(2, 2)
        else:
            idx_wait(i2, i2, b2)
            addcn(i2)
            gather_start(i2, b2)

    def group(g, c):
        i0 = g * 3
        gpar = g & 1
        npar = 1 - gpar
        for b in range(3):
            i = i0 + b
            islot = b + 3 * gpar
            gather_wait(islot, b)
            idx_start(i + 3, b + 3 * npar, b)
            scale(islot, b)
            scatter_start(islot, b)
            b2 = (b + 2) % 3
            scatter_wait((b2 + 3 * gpar) if b > 0 else (b2 + 3 * npar), b2)
            if b < 1:
                islot2 = b + 2 + 3 * gpar
            else:
                islot2 = (b + 2) % 3 + 3 * npar
            idx_wait(i + 2, islot2, b2)
            addcn(islot2)
            gather_start(islot2, b2)
        return c

    lax.fori_loop(1, NSCH // 3 - 1, group, 0)

    gather_wait(3, 0)
    scale(3, 0)
    scatter_start(3, 0)
    scatter_wait(2 + 3 * 0, 2)
    idx_wait(NSCH - 1, 5, 2)
    addcn(5)
    gather_start(5, 2)
    for b in (1, 2):
        gather_wait(3 + b, b)
        scale(3 + b, b)
        scatter_start(3 + b, b)
        scatter_wait(3 + b - 1, b - 1)
    scatter_wait(5, 2)

    plsc.subcore_barrier()
    pltpu.sync_copy(acc.at[pl.ds(base_r, RPS)],
                    out_hbm.at[cid, pl.ds(base_r, RPS)])


def _spmm(s2, src, dst, ew):
    return _spmm_sc(s2.reshape(2 * N, 128), src, dst, ew)


def kernel(x, edge_index, edge_weights, W1, b1, W2, b2, W3, b3, W4, b4, Wl, bl):
    e = edge_index.shape[1]
    pad = EPAD - e
    src = jnp.concatenate([edge_index[0], jnp.zeros((pad,), jnp.int32)])
    dst = jnp.concatenate([edge_index[1], jnp.zeros((pad,), jnp.int32)])
    ew = jnp.concatenate([edge_weights, jnp.zeros((pad,), jnp.float32)])

    dinv = _dinv(_deg_partials(dst, ew))
    s2 = _t0(x, W1, dinv)
    for b, W in ((b1, W2), (b2, W3), (b3, W4)):
        agg2 = _spmm(s2, src, dst, ew)
        s2 = _tmid(agg2, s2, dinv, b, W)
    agg2 = _spmm(s2, src, dst, ew)
    return _tfin(agg2, s2, dinv, b4, Wl, bl)
